# Initial kernel scaffold; baseline (speedup 1.0000x reference)
import jax
import jax.numpy as jnp
from jax.experimental import pallas as pl


def kernel(hidden_activation, slot_i):
    raise NotImplementedError("write your pallas kernel here")


# speedup vs baseline: 10.1982x; 10.1982x over previous
import jax
import jax.numpy as jnp
from jax import lax
from jax.experimental import pallas as pl
from jax.experimental.pallas import tpu as pltpu

_BR = 128


def _body(slot_ref, out_ref):
    i = pl.program_id(0)
    br, h = out_ref.shape
    slot_blk = slot_ref[pl.ds(i * br, br)]
    col = lax.broadcasted_iota(jnp.int32, (br, h), 1)
    out_ref[...] = (col == slot_blk[:, None]).astype(jnp.float32)


def kernel(hidden_activation, slot_i):
    b, h = hidden_activation.shape
    grid = (b // _BR,)
    return pl.pallas_call(
        _body,
        grid=grid,
        in_specs=[pl.BlockSpec((b,), lambda i: (0,))],
        out_specs=pl.BlockSpec((_BR, h), lambda i: (i, 0)),
        out_shape=jax.ShapeDtypeStruct((b, h), jnp.float32),
    )(slot_i)
